# baseline (speedup 1.0000x reference)
import jax
import jax.numpy as jnp
import numpy as np
from jax import lax
from jax.experimental import pallas as pl
from jax.experimental.pallas import tpu as pltpu
from jax.experimental.pallas import tpu_sc as plsc

N_ATOMS = 50000
NUM_TYPES = 64
_IRREPS = [(128, 1), (64, 3), (32, 5)]
NUM_SCALAR = 128
DIM = sum(m * d for m, d in _IRREPS)
NUM_IRREPS = sum(m for m, _ in _IRREPS)

_scale_idx = []
_k = 0
for _mul, _irdim in _IRREPS:
    for _ in range(_mul):
        _scale_idx += [_k] * _irdim
        _k += 1
SCALE_IDX_NP = np.asarray(_scale_idx, dtype=np.int32)
assert SCALE_IDX_NP.shape[0] == DIM

NC, NS, L = 2, 16, 16
NW = NC * NS
T = 16
NTILES = N_ATOMS // T
assert N_ATOMS % T == 0

_COLS30 = DIM // L


def _body(nf_hbm, at_hbm, scales_hbm, shifts_hbm, sidx_hbm, out_hbm,
          scales_v, shifts_v, sidx_v, table_v, nf_v, out_v, spec_v, sem):
    wid = lax.axis_index("s") * NC + lax.axis_index("c")

    pltpu.sync_copy(scales_hbm, scales_v)
    pltpu.sync_copy(shifts_hbm, shifts_v)
    pltpu.sync_copy(sidx_hbm, sidx_v)

    iota = lax.iota(jnp.int32, L)
    i480 = iota * DIM

    def expand_row(r, carry):
        rbase = jnp.full((L,), r * NUM_IRREPS, dtype=jnp.int32)
        for cb in range(_COLS30):
            col = sidx_v[pl.ds(cb * L, L)]
            vals = plsc.load_gather(scales_v, [rbase + col])
            table_v[pl.ds(r * DIM + cb * L, L)] = vals
        return carry

    lax.fori_loop(0, NUM_TYPES, expand_row, 0)

    ntw = (NTILES - 1 - wid) // NW + 1

    def do_tile(g, carry):
        base = (wid + g * NW) * T
        pltpu.sync_copy(nf_hbm.at[pl.ds(base * DIM, T * DIM)], nf_v)
        pltpu.sync_copy(at_hbm.at[pl.ds(base, T)], spec_v)
        s16 = spec_v[...]
        s480 = s16 * DIM
        s128 = s16 * NUM_SCALAR

        @plsc.parallel_loop(0, NUM_SCALAR, 1, unroll=4)
        def col_shift(c):
            cc = jnp.full((L,), c, dtype=jnp.int32)
            sc = plsc.load_gather(table_v, [s480 + cc])
            nfv = plsc.load_gather(nf_v, [i480 + cc])
            sh = plsc.load_gather(shifts_v, [s128 + cc])
            plsc.store_scatter(out_v, [i480 + cc], nfv * sc + sh)

        @plsc.parallel_loop(NUM_SCALAR, DIM, 1, unroll=4)
        def col_noshift(c):
            cc = jnp.full((L,), c, dtype=jnp.int32)
            sc = plsc.load_gather(table_v, [s480 + cc])
            nfv = plsc.load_gather(nf_v, [i480 + cc])
            plsc.store_scatter(out_v, [i480 + cc], nfv * sc)

        pltpu.sync_copy(out_v, out_hbm.at[pl.ds(base * DIM, T * DIM)])
        return carry

    lax.fori_loop(0, ntw, do_tile, 0)


@jax.jit
def _run(nf_flat, at, scales_flat, shifts_flat, sidx):
    mesh = plsc.VectorSubcoreMesh(core_axis_name="c", subcore_axis_name="s")
    f = pl.kernel(
        _body,
        out_type=jax.ShapeDtypeStruct((N_ATOMS * DIM,), jnp.float32),
        mesh=mesh,
        compiler_params=pltpu.CompilerParams(needs_layout_passes=False),
        scratch_types=[
            pltpu.VMEM((NUM_TYPES * NUM_IRREPS,), jnp.float32),
            pltpu.VMEM((NUM_TYPES * NUM_SCALAR,), jnp.float32),
            pltpu.VMEM((DIM,), jnp.int32),
            pltpu.VMEM((NUM_TYPES * DIM,), jnp.float32),
            pltpu.VMEM((T * DIM,), jnp.float32),
            pltpu.VMEM((T * DIM,), jnp.float32),
            pltpu.VMEM((T,), jnp.int32),
            pltpu.SemaphoreType.DMA,
        ],
    )
    out = f(nf_flat, at, scales_flat, shifts_flat, sidx)
    return out.reshape(N_ATOMS, DIM)


def kernel(node_features, atom_types, scales, shifts):
    sidx = jnp.asarray(SCALE_IDX_NP)
    return _run(node_features.reshape(-1), atom_types.astype(jnp.int32),
                scales.reshape(-1), shifts.reshape(-1), sidx)


# speedup vs baseline: 1.1037x; 1.1037x over previous
import jax
import jax.numpy as jnp
import numpy as np
from jax import lax
from jax.experimental import pallas as pl
from jax.experimental.pallas import tpu as pltpu
from jax.experimental.pallas import tpu_sc as plsc

N_ATOMS = 50000
NUM_TYPES = 64
_IRREPS = [(128, 1), (64, 3), (32, 5)]
NUM_SCALAR = 128
DIM = sum(m * d for m, d in _IRREPS)
NUM_IRREPS = sum(m for m, _ in _IRREPS)

_scale_idx = []
_k = 0
for _mul, _irdim in _IRREPS:
    for _ in range(_mul):
        _scale_idx += [_k] * _irdim
        _k += 1
SCALE_IDX_NP = np.asarray(_scale_idx, dtype=np.int32)
assert SCALE_IDX_NP.shape[0] == DIM

NC, NS, L = 2, 16, 16
NW = NC * NS
T = 16
NTILES = N_ATOMS // T
assert N_ATOMS % T == 0

_COLS30 = DIM // L


def _body(nf_hbm, at_hbm, scales_hbm, shifts_hbm, sidx_hbm, out_hbm,
          scales_v, shifts_v, sidx_v, table_v,
          nf0, nf1, out0, out1, sp0, sp1,
          isem0, isem1, osem0, osem1):
    wid = lax.axis_index("s") * NC + lax.axis_index("c")

    pltpu.sync_copy(scales_hbm, scales_v)
    pltpu.sync_copy(shifts_hbm, shifts_v)
    pltpu.sync_copy(sidx_hbm, sidx_v)

    iota = lax.iota(jnp.int32, L)
    i480 = iota * DIM

    def expand_row(r, carry):
        rbase = jnp.full((L,), r * NUM_IRREPS, dtype=jnp.int32)
        for cb in range(_COLS30):
            col = sidx_v[pl.ds(cb * L, L)]
            vals = plsc.load_gather(scales_v, [rbase + col])
            table_v[pl.ds(r * DIM + cb * L, L)] = vals
        return carry

    lax.fori_loop(0, NUM_TYPES, expand_row, 0)

    ntw = (NTILES - 1 - wid) // NW + 1

    def tbase(t):
        return (wid + t * NW) * T

    def issue_in(t, nf_v, sp_v, isem):
        b = tbase(t)
        pltpu.async_copy(nf_hbm.at[pl.ds(b * DIM, T * DIM)], nf_v, isem)
        pltpu.async_copy(at_hbm.at[pl.ds(b, T)], sp_v, isem)

    def compute(nf_v, sp_v, out_v):
        s16 = sp_v[...]
        s480 = s16 * DIM
        s128 = s16 * NUM_SCALAR

        @plsc.parallel_loop(0, NUM_SCALAR, 1, unroll=16)
        def col_shift(c):
            cc = jnp.full((L,), c, dtype=jnp.int32)
            sc = plsc.load_gather(table_v, [s480 + cc])
            nfv = plsc.load_gather(nf_v, [i480 + cc])
            sh = plsc.load_gather(shifts_v, [s128 + cc])
            plsc.store_scatter(out_v, [i480 + cc], nfv * sc + sh)

        @plsc.parallel_loop(NUM_SCALAR, DIM, 1, unroll=16)
        def col_noshift(c):
            cc = jnp.full((L,), c, dtype=jnp.int32)
            sc = plsc.load_gather(table_v, [s480 + cc])
            nfv = plsc.load_gather(nf_v, [i480 + cc])
            plsc.store_scatter(out_v, [i480 + cc], nfv * sc)

    def slot(t, h, nf_v, sp_v, out_v, isem, osem):
        b = tbase(t)
        pltpu.make_async_copy(nf_hbm.at[pl.ds(b * DIM, T * DIM)], nf_v,
                              isem).wait()
        pltpu.make_async_copy(at_hbm.at[pl.ds(b, T)], sp_v, isem).wait()

        @pl.when(h > 0)
        def _():
            pltpu.make_async_copy(out_v, out_hbm.at[pl.ds(b * DIM, T * DIM)],
                                  osem).wait()

        compute(nf_v, sp_v, out_v)
        pltpu.async_copy(out_v, out_hbm.at[pl.ds(b * DIM, T * DIM)], osem)

        @pl.when(t + 2 < ntw)
        def _():
            issue_in(t + 2, nf_v, sp_v, isem)

    issue_in(0, nf0, sp0, isem0)
    issue_in(1, nf1, sp1, isem1)

    nh = (ntw + 1) // 2

    def pair(h, carry):
        slot(2 * h, h, nf0, sp0, out0, isem0, osem0)

        @pl.when(2 * h + 1 < ntw)
        def _():
            slot(2 * h + 1, h, nf1, sp1, out1, isem1, osem1)

        return carry

    lax.fori_loop(0, nh, pair, 0)

    pltpu.make_async_copy(out0, out_hbm.at[pl.ds(0, T * DIM)], osem0).wait()
    pltpu.make_async_copy(out1, out_hbm.at[pl.ds(0, T * DIM)], osem1).wait()


@jax.jit
def _run(nf_flat, at, scales_flat, shifts_flat, sidx):
    mesh = plsc.VectorSubcoreMesh(core_axis_name="c", subcore_axis_name="s")
    f = pl.kernel(
        _body,
        out_type=jax.ShapeDtypeStruct((N_ATOMS * DIM,), jnp.float32),
        mesh=mesh,
        compiler_params=pltpu.CompilerParams(needs_layout_passes=False),
        scratch_types=[
            pltpu.VMEM((NUM_TYPES * NUM_IRREPS,), jnp.float32),
            pltpu.VMEM((NUM_TYPES * NUM_SCALAR,), jnp.float32),
            pltpu.VMEM((DIM,), jnp.int32),
            pltpu.VMEM((NUM_TYPES * DIM,), jnp.float32),
            pltpu.VMEM((T * DIM,), jnp.float32),
            pltpu.VMEM((T * DIM,), jnp.float32),
            pltpu.VMEM((T * DIM,), jnp.float32),
            pltpu.VMEM((T * DIM,), jnp.float32),
            pltpu.VMEM((T,), jnp.int32),
            pltpu.VMEM((T,), jnp.int32),
            pltpu.SemaphoreType.DMA,
            pltpu.SemaphoreType.DMA,
            pltpu.SemaphoreType.DMA,
            pltpu.SemaphoreType.DMA,
        ],
    )
    out = f(nf_flat, at, scales_flat, shifts_flat, sidx)
    return out.reshape(N_ATOMS, DIM)


def kernel(node_features, atom_types, scales, shifts):
    sidx = jnp.asarray(SCALE_IDX_NP)
    return _run(node_features.reshape(-1), atom_types.astype(jnp.int32),
                scales.reshape(-1), shifts.reshape(-1), sidx)


# speedup vs baseline: 1.7436x; 1.5798x over previous
import jax
import jax.numpy as jnp
import numpy as np
from jax import lax
from jax.experimental import pallas as pl
from jax.experimental.pallas import tpu as pltpu
from jax.experimental.pallas import tpu_sc as plsc

N_ATOMS = 50000
NUM_TYPES = 64
_IRREPS = [(128, 1), (64, 3), (32, 5)]
NUM_SCALAR = 128
DIM = sum(m * d for m, d in _IRREPS)
NUM_IRREPS = sum(m for m, _ in _IRREPS)

_scale_idx = []
_k = 0
for _mul, _irdim in _IRREPS:
    for _ in range(_mul):
        _scale_idx += [_k] * _irdim
        _k += 1
SCALE_IDX_NP = np.asarray(_scale_idx, dtype=np.int32)
assert SCALE_IDX_NP.shape[0] == DIM

NC, NS, L = 2, 16, 16
NW = NC * NS
T = 16
NTILES = N_ATOMS // T
assert N_ATOMS % T == 0

_COLS30 = DIM // L


def _body(nf_hbm, at_hbm, scales_hbm, shifts_hbm, sidx_hbm, out_hbm,
          scales_v, shifts_v, sidx_v, table_v,
          nf0, nf1, out0, out1, sp0, sp1,
          isem0, isem1, osem0, osem1):
    wid = lax.axis_index("s") * NC + lax.axis_index("c")

    pltpu.sync_copy(scales_hbm, scales_v)
    pltpu.sync_copy(shifts_hbm, shifts_v)
    pltpu.sync_copy(sidx_hbm, sidx_v)

    iota = lax.iota(jnp.int32, L)
    i480 = iota * DIM

    def expand_row(r, carry):
        rbase = jnp.full((L,), r * NUM_IRREPS, dtype=jnp.int32)
        for cb in range(_COLS30):
            col = sidx_v[pl.ds(cb * L, L)]
            vals = plsc.load_gather(scales_v, [rbase + col])
            table_v[pl.ds(r * DIM + cb * L, L)] = vals
        return carry

    lax.fori_loop(0, NUM_TYPES, expand_row, 0)

    ntw = (NTILES - 1 - wid) // NW + 1

    def tbase(t):
        return (wid + t * NW) * T

    def issue_in(t, nf_v, sp_v, isem):
        b = tbase(t)
        pltpu.async_copy(nf_hbm.at[pl.ds(b * DIM, T * DIM)], nf_v, isem)
        pltpu.async_copy(at_hbm.at[pl.ds(b, T)], sp_v, isem)

    def compute(nf_v, sp_v, out_v):
        @plsc.parallel_loop(0, T, 1)
        def per_atom(a):
            av = jnp.full((L,), a, dtype=jnp.int32)
            sa = plsc.load_gather(sp_v, [av])
            sbase = sa * DIM + iota
            hbase = sa * NUM_SCALAR + iota
            abase = a * DIM
            for cb in range(NUM_SCALAR // L):
                nfc = nf_v[pl.ds(abase + cb * L, L)]
                scc = plsc.load_gather(table_v, [sbase + (cb * L)])
                shc = plsc.load_gather(shifts_v, [hbase + (cb * L)])
                out_v[pl.ds(abase + cb * L, L)] = nfc * scc + shc
            for cb in range(NUM_SCALAR // L, _COLS30):
                nfc = nf_v[pl.ds(abase + cb * L, L)]
                scc = plsc.load_gather(table_v, [sbase + (cb * L)])
                out_v[pl.ds(abase + cb * L, L)] = nfc * scc

    def slot(t, h, nf_v, sp_v, out_v, isem, osem):
        b = tbase(t)
        pltpu.make_async_copy(nf_hbm.at[pl.ds(b * DIM, T * DIM)], nf_v,
                              isem).wait()
        pltpu.make_async_copy(at_hbm.at[pl.ds(b, T)], sp_v, isem).wait()

        @pl.when(h > 0)
        def _():
            pltpu.make_async_copy(out_v, out_hbm.at[pl.ds(b * DIM, T * DIM)],
                                  osem).wait()

        compute(nf_v, sp_v, out_v)
        pltpu.async_copy(out_v, out_hbm.at[pl.ds(b * DIM, T * DIM)], osem)

        @pl.when(t + 2 < ntw)
        def _():
            issue_in(t + 2, nf_v, sp_v, isem)

    issue_in(0, nf0, sp0, isem0)
    issue_in(1, nf1, sp1, isem1)

    nh = (ntw + 1) // 2

    def pair(h, carry):
        slot(2 * h, h, nf0, sp0, out0, isem0, osem0)

        @pl.when(2 * h + 1 < ntw)
        def _():
            slot(2 * h + 1, h, nf1, sp1, out1, isem1, osem1)

        return carry

    lax.fori_loop(0, nh, pair, 0)

    pltpu.make_async_copy(out0, out_hbm.at[pl.ds(0, T * DIM)], osem0).wait()
    pltpu.make_async_copy(out1, out_hbm.at[pl.ds(0, T * DIM)], osem1).wait()


@jax.jit
def _run(nf_flat, at, scales_flat, shifts_flat, sidx):
    mesh = plsc.VectorSubcoreMesh(core_axis_name="c", subcore_axis_name="s")
    f = pl.kernel(
        _body,
        out_type=jax.ShapeDtypeStruct((N_ATOMS * DIM,), jnp.float32),
        mesh=mesh,
        compiler_params=pltpu.CompilerParams(needs_layout_passes=False),
        scratch_types=[
            pltpu.VMEM((NUM_TYPES * NUM_IRREPS,), jnp.float32),
            pltpu.VMEM((NUM_TYPES * NUM_SCALAR,), jnp.float32),
            pltpu.VMEM((DIM,), jnp.int32),
            pltpu.VMEM((NUM_TYPES * DIM,), jnp.float32),
            pltpu.VMEM((T * DIM,), jnp.float32),
            pltpu.VMEM((T * DIM,), jnp.float32),
            pltpu.VMEM((T * DIM,), jnp.float32),
            pltpu.VMEM((T * DIM,), jnp.float32),
            pltpu.VMEM((T,), jnp.int32),
            pltpu.VMEM((T,), jnp.int32),
            pltpu.SemaphoreType.DMA,
            pltpu.SemaphoreType.DMA,
            pltpu.SemaphoreType.DMA,
            pltpu.SemaphoreType.DMA,
        ],
    )
    out = f(nf_flat, at, scales_flat, shifts_flat, sidx)
    return out.reshape(N_ATOMS, DIM)


def kernel(node_features, atom_types, scales, shifts):
    sidx = jnp.asarray(SCALE_IDX_NP)
    return _run(node_features.reshape(-1), atom_types.astype(jnp.int32),
                scales.reshape(-1), shifts.reshape(-1), sidx)


# speedup vs baseline: 6.6094x; 3.7907x over previous
import jax
import jax.numpy as jnp
import numpy as np
from jax import lax
from jax.experimental import pallas as pl
from jax.experimental.pallas import tpu as pltpu
from jax.experimental.pallas import tpu_sc as plsc

N_ATOMS = 50000
NUM_TYPES = 64
_IRREPS = [(128, 1), (64, 3), (32, 5)]
NUM_SCALAR = 128
DIM = sum(m * d for m, d in _IRREPS)
NUM_IRREPS = sum(m for m, _ in _IRREPS)

_scale_idx = []
_k = 0
for _mul, _irdim in _IRREPS:
    for _ in range(_mul):
        _scale_idx += [_k] * _irdim
        _k += 1
SCALE_IDX_NP = np.asarray(_scale_idx, dtype=np.int32)
assert SCALE_IDX_NP.shape[0] == DIM

NC, NS, L = 2, 16, 16
NW = NC * NS
T = 16
NTILES = N_ATOMS // T
assert N_ATOMS % T == 0

_COLS30 = DIM // L


def _body(nf_hbm, at_hbm, scales_hbm, shifts_hbm, sidx_hbm, out_hbm,
          scales_v, shifts_v, sidx_v, table_v,
          nf0, nf1, out0, out1, sp0, sp1,
          isem0, isem1, osem0, osem1):
    wid = lax.axis_index("s") * NC + lax.axis_index("c")

    pltpu.sync_copy(scales_hbm, scales_v)
    pltpu.sync_copy(shifts_hbm, shifts_v)
    pltpu.sync_copy(sidx_hbm, sidx_v)

    iota = lax.iota(jnp.int32, L)

    def expand_row(r, carry):
        rbase = jnp.full((L,), r * NUM_IRREPS, dtype=jnp.int32)
        for cb in range(_COLS30):
            col = sidx_v[pl.ds(cb * L, L)]
            vals = plsc.load_gather(scales_v, [rbase + col])
            table_v[pl.ds(r * DIM + cb * L, L)] = vals
        return carry

    lax.fori_loop(0, NUM_TYPES, expand_row, 0)

    ntw = (NTILES - 1 - wid) // NW + 1

    def tbase(t):
        return (wid + t * NW) * T

    def issue_in(t, nf_v, sp_v, isem):
        b = tbase(t)
        pltpu.async_copy(nf_hbm.at[pl.ds(b, T), :], nf_v, isem)
        pltpu.async_copy(at_hbm.at[pl.ds(b, T)], sp_v, isem)

    def compute(nf_v, sp_v, out_v):
        @plsc.parallel_loop(0, T, 1)
        def per_atom(a):
            av = jnp.full((L,), a, dtype=jnp.int32)
            sa = plsc.load_gather(sp_v, [av])
            sbase = sa * DIM + iota
            hbase = sa * NUM_SCALAR + iota
            for cb in range(NUM_SCALAR // L):
                nfc = nf_v[a, pl.ds(cb * L, L)]
                scc = plsc.load_gather(table_v, [sbase + (cb * L)])
                shc = plsc.load_gather(shifts_v, [hbase + (cb * L)])
                out_v[a, pl.ds(cb * L, L)] = nfc * scc + shc
            for cb in range(NUM_SCALAR // L, _COLS30):
                nfc = nf_v[a, pl.ds(cb * L, L)]
                scc = plsc.load_gather(table_v, [sbase + (cb * L)])
                out_v[a, pl.ds(cb * L, L)] = nfc * scc

    def slot(t, h, nf_v, sp_v, out_v, isem, osem):
        b = tbase(t)
        pltpu.make_async_copy(nf_hbm.at[pl.ds(b, T), :], nf_v, isem).wait()
        pltpu.make_async_copy(at_hbm.at[pl.ds(b, T)], sp_v, isem).wait()

        @pl.when(h > 0)
        def _():
            pltpu.make_async_copy(out_v, out_hbm.at[pl.ds(b, T), :],
                                  osem).wait()

        compute(nf_v, sp_v, out_v)
        pltpu.async_copy(out_v, out_hbm.at[pl.ds(b, T), :], osem)

        @pl.when(t + 2 < ntw)
        def _():
            issue_in(t + 2, nf_v, sp_v, isem)

    issue_in(0, nf0, sp0, isem0)
    issue_in(1, nf1, sp1, isem1)

    nh = (ntw + 1) // 2

    def pair(h, carry):
        slot(2 * h, h, nf0, sp0, out0, isem0, osem0)

        @pl.when(2 * h + 1 < ntw)
        def _():
            slot(2 * h + 1, h, nf1, sp1, out1, isem1, osem1)

        return carry

    lax.fori_loop(0, nh, pair, 0)

    pltpu.make_async_copy(out0, out_hbm.at[pl.ds(0, T), :], osem0).wait()
    pltpu.make_async_copy(out1, out_hbm.at[pl.ds(0, T), :], osem1).wait()


@jax.jit
def _run(nf, at, scales_flat, shifts_flat, sidx):
    mesh = plsc.VectorSubcoreMesh(core_axis_name="c", subcore_axis_name="s")
    f = pl.kernel(
        _body,
        out_type=jax.ShapeDtypeStruct((N_ATOMS, DIM), jnp.float32),
        mesh=mesh,
        compiler_params=pltpu.CompilerParams(needs_layout_passes=False),
        scratch_types=[
            pltpu.VMEM((NUM_TYPES * NUM_IRREPS,), jnp.float32),
            pltpu.VMEM((NUM_TYPES * NUM_SCALAR,), jnp.float32),
            pltpu.VMEM((DIM,), jnp.int32),
            pltpu.VMEM((NUM_TYPES * DIM,), jnp.float32),
            pltpu.VMEM((T, DIM), jnp.float32),
            pltpu.VMEM((T, DIM), jnp.float32),
            pltpu.VMEM((T, DIM), jnp.float32),
            pltpu.VMEM((T, DIM), jnp.float32),
            pltpu.VMEM((T,), jnp.int32),
            pltpu.VMEM((T,), jnp.int32),
            pltpu.SemaphoreType.DMA,
            pltpu.SemaphoreType.DMA,
            pltpu.SemaphoreType.DMA,
            pltpu.SemaphoreType.DMA,
        ],
    )
    return f(nf, at, scales_flat, shifts_flat, sidx)


def kernel(node_features, atom_types, scales, shifts):
    sidx = jnp.asarray(SCALE_IDX_NP)
    return _run(node_features, atom_types.astype(jnp.int32),
                scales.reshape(-1), shifts.reshape(-1), sidx)


# speedup vs baseline: 6.6183x; 1.0014x over previous
import jax
import jax.numpy as jnp
import numpy as np
from jax import lax
from jax.experimental import pallas as pl
from jax.experimental.pallas import tpu as pltpu
from jax.experimental.pallas import tpu_sc as plsc

N_ATOMS = 50000
NUM_TYPES = 64
_IRREPS = [(128, 1), (64, 3), (32, 5)]
NUM_SCALAR = 128
DIM = sum(m * d for m, d in _IRREPS)
NUM_IRREPS = sum(m for m, _ in _IRREPS)

_scale_idx = []
_k = 0
for _mul, _irdim in _IRREPS:
    for _ in range(_mul):
        _scale_idx += [_k] * _irdim
        _k += 1
SCALE_IDX_NP = np.asarray(_scale_idx, dtype=np.int32)
assert SCALE_IDX_NP.shape[0] == DIM

NC, NS, L = 2, 16, 16
NW = NC * NS
T = 16
NTILES = N_ATOMS // T
assert N_ATOMS % T == 0

_COLS30 = DIM // L


def _body(nf_hbm, at_hbm, scales_hbm, shifts_hbm, sidx_hbm, out_hbm,
          scales_v, shifts_v, sidx_v, table_v,
          nf0, nf1, out0, out1, sp0, sp1,
          isem0, isem1, osem0, osem1):
    wid = lax.axis_index("s") * NC + lax.axis_index("c")

    pltpu.sync_copy(scales_hbm, scales_v)
    pltpu.sync_copy(shifts_hbm, shifts_v)
    pltpu.sync_copy(sidx_hbm, sidx_v)

    iota = lax.iota(jnp.int32, L)

    def expand_row(r, carry):
        rbase = jnp.full((L,), r * NUM_IRREPS, dtype=jnp.int32)
        for cb in range(_COLS30):
            col = sidx_v[pl.ds(cb * L, L)]
            vals = plsc.load_gather(scales_v, [rbase + col])
            table_v[pl.ds(r * DIM + cb * L, L)] = vals
        return carry

    lax.fori_loop(0, NUM_TYPES, expand_row, 0)

    ntw = (NTILES - 1 - wid) // NW + 1

    def tbase(t):
        return (wid + t * NW) * T

    def issue_in(t, nf_v, sp_v, isem):
        b = tbase(t)
        pltpu.async_copy(nf_hbm.at[pl.ds(b, T), :], nf_v, isem)
        pltpu.async_copy(at_hbm.at[pl.ds(b, T)], sp_v, isem)

    def compute(nf_v, sp_v, out_v):
        @plsc.parallel_loop(0, T, 1)
        def per_atom(a):
            av = jnp.full((L,), a, dtype=jnp.int32)
            sa = plsc.load_gather(sp_v, [av])
            sbase = sa * DIM + iota
            hbase = sa * NUM_SCALAR + iota
            for cb in range(NUM_SCALAR // L):
                nfc = nf_v[a, pl.ds(cb * L, L)]
                scc = plsc.load_gather(table_v, [sbase + (cb * L)])
                shc = plsc.load_gather(shifts_v, [hbase + (cb * L)])
                out_v[a, pl.ds(cb * L, L)] = nfc * scc + shc
            for cb in range(NUM_SCALAR // L, _COLS30):
                nfc = nf_v[a, pl.ds(cb * L, L)]
                scc = plsc.load_gather(table_v, [sbase + (cb * L)])
                out_v[a, pl.ds(cb * L, L)] = nfc * scc

    def slot(t, h, nf_v, sp_v, out_v, isem, osem):
        b = tbase(t)
        pltpu.make_async_copy(nf_hbm.at[pl.ds(b, T), :], nf_v, isem).wait()
        pltpu.make_async_copy(at_hbm.at[pl.ds(b, T)], sp_v, isem).wait()

        @pl.when(h > 0)
        def _():
            pltpu.make_async_copy(out_v, out_hbm.at[pl.ds(b, T), :],
                                  osem).wait()

        compute(nf_v, sp_v, out_v)
        pltpu.async_copy(out_v, out_hbm.at[pl.ds(b, T), :], osem)

        @pl.when(t + 2 < ntw)
        def _():
            issue_in(t + 2, nf_v, sp_v, isem)

    issue_in(0, nf0, sp0, isem0)
    issue_in(1, nf1, sp1, isem1)

    nh = (ntw + 1) // 2

    def pair(h, carry):
        slot(2 * h, h, nf0, sp0, out0, isem0, osem0)

        @pl.when(2 * h + 1 < ntw)
        def _():
            slot(2 * h + 1, h, nf1, sp1, out1, isem1, osem1)

        return carry

    lax.fori_loop(0, nh, pair, 0)

    pltpu.make_async_copy(out0, out_hbm.at[pl.ds(0, T), :], osem0).wait()
    pltpu.make_async_copy(out1, out_hbm.at[pl.ds(0, T), :], osem1).wait()


@jax.jit
def _run(nf, at, scales_flat, shifts_flat, sidx):
    mesh = plsc.VectorSubcoreMesh(core_axis_name="c", subcore_axis_name="s")
    f = pl.kernel(
        _body,
        out_type=jax.ShapeDtypeStruct((N_ATOMS, DIM), jnp.float32),
        mesh=mesh,
        compiler_params=pltpu.CompilerParams(needs_layout_passes=False,
                                             use_tc_tiling_on_sc=True),
        scratch_types=[
            pltpu.VMEM((NUM_TYPES * NUM_IRREPS,), jnp.float32),
            pltpu.VMEM((NUM_TYPES * NUM_SCALAR,), jnp.float32),
            pltpu.VMEM((DIM,), jnp.int32),
            pltpu.VMEM((NUM_TYPES * DIM,), jnp.float32),
            pltpu.VMEM((T, DIM), jnp.float32),
            pltpu.VMEM((T, DIM), jnp.float32),
            pltpu.VMEM((T, DIM), jnp.float32),
            pltpu.VMEM((T, DIM), jnp.float32),
            pltpu.VMEM((T,), jnp.int32),
            pltpu.VMEM((T,), jnp.int32),
            pltpu.SemaphoreType.DMA,
            pltpu.SemaphoreType.DMA,
            pltpu.SemaphoreType.DMA,
            pltpu.SemaphoreType.DMA,
        ],
    )
    return f(nf, at, scales_flat, shifts_flat, sidx)


def kernel(node_features, atom_types, scales, shifts):
    sidx = jnp.asarray(SCALE_IDX_NP)
    return _run(node_features, atom_types.astype(jnp.int32),
                scales.reshape(-1), shifts.reshape(-1), sidx)
